# baseline (speedup 1.0000x reference)
import jax
import jax.numpy as jnp
from jax.experimental import pallas as pl


def _copy_kernel(in_ref, out_ref):
    out_ref[...] = in_ref[...]


def _blocked_copy(x, block_rows):
    n, d = x.shape
    grid = pl.cdiv(n, block_rows)
    return pl.pallas_call(
        _copy_kernel,
        grid=(grid,),
        in_specs=[pl.BlockSpec((block_rows, d), lambda i: (i, 0))],
        out_specs=pl.BlockSpec((block_rows, d), lambda i: (i, 0)),
        out_shape=jax.ShapeDtypeStruct(x.shape, x.dtype),
    )(x)


def kernel(emb_user, emb_item):
    u = emb_user.reshape(500_000, 128)
    it = emb_item.reshape(50_000, 128)
    out_u = _blocked_copy(u, 5_000).reshape(emb_user.shape)
    out_it = _blocked_copy(it, 5_000).reshape(emb_item.shape)
    return (out_u, out_it)
